# Initial kernel scaffold; baseline (speedup 1.0000x reference)
import jax
import jax.numpy as jnp
from jax.experimental import pallas as pl


def kernel(word_seq, position_enc_weight):
    raise NotImplementedError("write your pallas kernel here")


# speedup vs baseline: 1.0665x; 1.0665x over previous
import functools

import jax
import jax.numpy as jnp
from jax import lax
from jax.experimental import pallas as pl
from jax.experimental.pallas import tpu as pltpu
from jax.experimental.pallas import tpu_sc as plsc

EMB = 64
SEQ = 200
SEQ_PAD = 208
NCHUNK = 13
BATCH = 4096
NWORKERS = 32
ROWS_PER_W = BATCH // NWORKERS
GA = 112
GB = 112


def _sc_body(seq_hbm, table_hbm, out_hbm, seq_v, idx_a, idx_b, rows_v, sem):
    cid = lax.axis_index("c")
    sid = lax.axis_index("s")
    wid = sid * 2 + cid
    base = wid * ROWS_PER_W

    zeros16 = jnp.zeros((16,), jnp.int32)
    ones16 = jnp.ones((16,), jnp.int32)
    lane = lax.iota(jnp.int32, 16)
    tail_valid = lane < jnp.full((16,), 8, jnp.int32)
    scan_idx = [jnp.maximum(lane - (1 << k), zeros16) for k in range(4)]
    scan_msk = [lane >= jnp.full((16,), 1 << k, jnp.int32) for k in range(4)]
    idx_last = jnp.full((16,), 15, jnp.int32)

    dnums = lax.GatherDimensionNumbers(
        offset_dims=(), collapsed_slice_dims=(0,), start_index_map=(0,))

    def _lanegather(x, idx):
        return lax.gather(x, idx[:, None], dnums, slice_sizes=(1,),
                          mode=lax.GatherScatterMode.PROMISE_IN_BOUNDS)

    def _cumsum16(m):
        s = m
        for k in range(4):
            g = _lanegather(s, scan_idx[k])
            s = s + jnp.where(scan_msk[k], g, zeros16)
        return s
    idx_b[pl.ds(96, 16)] = zeros16

    def row_loop(r, carry_unused):
        b = base + r
        pltpu.sync_copy(seq_hbm.at[pl.ds(b * SEQ, SEQ)], seq_v.at[pl.ds(0, SEQ)])
        carry = zeros16
        for c in range(NCHUNK):
            v = seq_v[pl.ds(16 * c, 16)]
            nz = v != zeros16
            if c == NCHUNK - 1:
                nz = jnp.logical_and(nz, tail_valid)
            m = jnp.where(nz, ones16, zeros16)
            s = _cumsum16(m)
            pos = (s + carry) * m
            if c < 7:
                idx_a[pl.ds(16 * c, 16)] = pos
            else:
                idx_b[pl.ds(16 * (c - 7), 16)] = pos
            carry = carry + _lanegather(s, idx_last)
        cp_a = pltpu.async_copy(table_hbm.at[idx_a], rows_v.at[pl.ds(0, GA)], sem)
        cp_b = pltpu.async_copy(table_hbm.at[idx_b], rows_v.at[pl.ds(GA, GB)], sem)
        cp_a.wait()
        cp_b.wait()
        pltpu.sync_copy(rows_v.at[pl.ds(0, SEQ)], out_hbm.at[pl.ds(b * SEQ, SEQ)])
        return carry_unused

    lax.fori_loop(0, ROWS_PER_W, row_loop, jnp.int32(0))


@functools.partial(jax.jit, static_argnames=())
def _sc_call(seq, table):
    fn = functools.partial(
        pl.kernel,
        mesh=plsc.VectorSubcoreMesh(core_axis_name="c", subcore_axis_name="s"),
        compiler_params=pltpu.CompilerParams(use_tc_tiling_on_sc=False),
        out_type=jax.ShapeDtypeStruct((BATCH * SEQ, EMB), jnp.float32),
        scratch_types=[
            pltpu.VMEM((SEQ_PAD,), jnp.int32),
            pltpu.VMEM((GA,), jnp.int32),
            pltpu.VMEM((GB,), jnp.int32),
            pltpu.VMEM((GA + GB, EMB), jnp.float32),
            pltpu.SemaphoreType.DMA,
        ],
    )(_sc_body)
    return fn(seq, table)


def kernel(word_seq, position_enc_weight):
    seq = word_seq.astype(jnp.int32).reshape(-1)
    out = _sc_call(seq, position_enc_weight)
    return out.reshape(BATCH, SEQ, EMB)


# speedup vs baseline: 1.0678x; 1.0012x over previous
import functools

import jax
import jax.numpy as jnp
from jax import lax
from jax.experimental import pallas as pl
from jax.experimental.pallas import tpu as pltpu
from jax.experimental.pallas import tpu_sc as plsc

EMB = 64
SEQ = 200
SEQ_PAD = 208
NCHUNK = 13
BATCH = 4096
NWORKERS = 32
ROWS_PER_W = BATCH // NWORKERS
GA = 112
GB = 112


def _sc_body(seq_hbm, table_hbm, out_hbm,
             seq0, seq1, ia0, ib0, ia1, ib1, rows0, rows1,
             sin0, sin1, sg0, sg1, sout0, sout1):
    cid = lax.axis_index("c")
    sid = lax.axis_index("s")
    wid = sid * 2 + cid
    base = wid * ROWS_PER_W

    zeros16 = jnp.zeros((16,), jnp.int32)
    ones16 = jnp.ones((16,), jnp.int32)
    lane = lax.iota(jnp.int32, 16)
    tail_valid = lane < jnp.full((16,), 8, jnp.int32)
    scan_idx = [jnp.maximum(lane - (1 << k), zeros16) for k in range(4)]
    scan_msk = [lane >= jnp.full((16,), 1 << k, jnp.int32) for k in range(4)]
    idx_last = jnp.full((16,), 15, jnp.int32)

    dnums = lax.GatherDimensionNumbers(
        offset_dims=(), collapsed_slice_dims=(0,), start_index_map=(0,))

    def _lanegather(x, idx):
        return lax.gather(x, idx[:, None], dnums, slice_sizes=(1,),
                          mode=lax.GatherScatterMode.PROMISE_IN_BOUNDS)

    def _cumsum16(m):
        s = m
        for k in range(4):
            g = _lanegather(s, scan_idx[k])
            s = s + jnp.where(scan_msk[k], g, zeros16)
        return s

    ib0[pl.ds(96, 16)] = zeros16
    ib1[pl.ds(96, 16)] = zeros16

    def in_cp(b, seq_v, sem):
        return pltpu.make_async_copy(
            seq_hbm.at[pl.ds(b * SEQ, SEQ)], seq_v.at[pl.ds(0, SEQ)], sem)

    def gather_cps(ia, ib, rows_v, sem):
        return (pltpu.make_async_copy(table_hbm.at[ia],
                                      rows_v.at[pl.ds(0, GA)], sem),
                pltpu.make_async_copy(table_hbm.at[ib],
                                      rows_v.at[pl.ds(GA, GB)], sem))

    def out_cp(b, rows_v, sem):
        return pltpu.make_async_copy(
            rows_v.at[pl.ds(0, SEQ)], out_hbm.at[pl.ds(b * SEQ, SEQ)], sem)

    def compute(seq_v, ia, ib):
        carry = zeros16
        for c in range(NCHUNK):
            v = seq_v[pl.ds(16 * c, 16)]
            nz = v != zeros16
            if c == NCHUNK - 1:
                nz = jnp.logical_and(nz, tail_valid)
            m = jnp.where(nz, ones16, zeros16)
            s = _cumsum16(m)
            pos = (s + carry) * m
            if c < 7:
                ia[pl.ds(16 * c, 16)] = pos
            else:
                ib[pl.ds(16 * (c - 7), 16)] = pos
            carry = carry + _lanegather(s, idx_last)

    bufs = ((seq0, ia0, ib0, rows0, sin0, sg0, sout0),
            (seq1, ia1, ib1, rows1, sin1, sg1, sout1))

    in_cp(base + 0, seq0, sin0).start()
    in_cp(base + 1, seq1, sin1).start()

    def pair_loop(r2, carry_unused):
        for p in (0, 1):
            r = 2 * r2 + p
            b = base + r
            seq_v, ia, ib, rows_v, sin, sg, sout = bufs[p]
            oseq_v, oia, oib, orows_v, osin, osg, osout = bufs[1 - p]
            in_cp(b, seq_v, sin).wait()
            compute(seq_v, ia, ib)

            @pl.when(r2 < (ROWS_PER_W // 2) - 1)
            def _():
                in_cp(b + 2, seq_v, sin).start()

            @pl.when(r2 >= 1)
            def _():
                out_cp(b - 2, rows_v, sout).wait()

            ca, cb = gather_cps(ia, ib, rows_v, sg)
            ca.start()
            cb.start()
            if p == 0:
                @pl.when(r2 >= 1)
                def _():
                    ga, gb = gather_cps(oia, oib, orows_v, osg)
                    ga.wait()
                    gb.wait()
                    out_cp(b - 1, orows_v, osout).start()
            else:
                ga, gb = gather_cps(oia, oib, orows_v, osg)
                ga.wait()
                gb.wait()
                out_cp(b - 1, orows_v, osout).start()
        return carry_unused

    lax.fori_loop(0, ROWS_PER_W // 2, pair_loop, jnp.int32(0))

    last = base + ROWS_PER_W - 1
    ga, gb = gather_cps(ia1, ib1, rows1, sg1)
    ga.wait()
    gb.wait()
    out_cp(last, rows1, sout1).start()
    out_cp(last - 1, rows0, sout0).wait()
    out_cp(last, rows1, sout1).wait()


@jax.jit
def _sc_call(seq, table):
    fn = functools.partial(
        pl.kernel,
        mesh=plsc.VectorSubcoreMesh(core_axis_name="c", subcore_axis_name="s"),
        compiler_params=pltpu.CompilerParams(use_tc_tiling_on_sc=False),
        out_type=jax.ShapeDtypeStruct((BATCH * SEQ, EMB), jnp.float32),
        scratch_types=[
            pltpu.VMEM((SEQ_PAD,), jnp.int32),
            pltpu.VMEM((SEQ_PAD,), jnp.int32),
            pltpu.VMEM((GA,), jnp.int32),
            pltpu.VMEM((GB,), jnp.int32),
            pltpu.VMEM((GA,), jnp.int32),
            pltpu.VMEM((GB,), jnp.int32),
            pltpu.VMEM((GA + GB, EMB), jnp.float32),
            pltpu.VMEM((GA + GB, EMB), jnp.float32),
            pltpu.SemaphoreType.DMA,
            pltpu.SemaphoreType.DMA,
            pltpu.SemaphoreType.DMA,
            pltpu.SemaphoreType.DMA,
            pltpu.SemaphoreType.DMA,
            pltpu.SemaphoreType.DMA,
        ],
    )(_sc_body)
    return fn(seq, table)


def kernel(word_seq, position_enc_weight):
    seq = word_seq.astype(jnp.int32).reshape(-1)
    out = _sc_call(seq, position_enc_weight)
    return out.reshape(BATCH, SEQ, EMB)


# speedup vs baseline: 4.7989x; 4.4943x over previous
import functools

import jax
import jax.numpy as jnp
from jax import lax
from jax.experimental import pallas as pl
from jax.experimental.pallas import tpu as pltpu
from jax.experimental.pallas import tpu_sc as plsc

EMB = 64
SEQ = 200
SEQ_PAD = 208
NCHUNK = 13
BATCH = 4096
NWORKERS = 32
ROWS_PER_W = BATCH // NWORKERS
GA = 112
GB = 112


def _sc_body(seq_hbm, table_hbm, out_hbm,
             seq0, seq1, ia0, ib0, ia1, ib1, rows0, rows1, tab_v, tot_v,
             sin0, sin1, sg0, sg1, sout0, sout1):
    cid = lax.axis_index("c")
    sid = lax.axis_index("s")
    wid = sid * 2 + cid
    base = wid * ROWS_PER_W

    zeros16 = jnp.zeros((16,), jnp.int32)
    ones16 = jnp.ones((16,), jnp.int32)
    lane = lax.iota(jnp.int32, 16)
    tail_valid = lane < jnp.full((16,), 8, jnp.int32)
    scan_idx = [jnp.maximum(lane - (1 << k), zeros16) for k in range(4)]
    scan_msk = [lane >= jnp.full((16,), 1 << k, jnp.int32) for k in range(4)]
    bfly_idx = [lane ^ jnp.full((16,), 1 << k, jnp.int32) for k in range(4)]
    idx_last = jnp.full((16,), 15, jnp.int32)

    dnums = lax.GatherDimensionNumbers(
        offset_dims=(), collapsed_slice_dims=(0,), start_index_map=(0,))

    def _lanegather(x, idx):
        return lax.gather(x, idx[:, None], dnums, slice_sizes=(1,),
                          mode=lax.GatherScatterMode.PROMISE_IN_BOUNDS)

    def _cumsum16(m):
        s = m
        for k in range(4):
            g = _lanegather(s, scan_idx[k])
            s = s + jnp.where(scan_msk[k], g, zeros16)
        return s

    def _allsum16(x):
        s = x
        for k in range(4):
            s = s + _lanegather(s, bfly_idx[k])
        return s

    ib0[pl.ds(96, 16)] = zeros16
    ib1[pl.ds(96, 16)] = zeros16

    def in_cp(b, seq_v, sem):
        return pltpu.make_async_copy(
            seq_hbm.at[pl.ds(b * SEQ, SEQ)], seq_v.at[pl.ds(0, SEQ)], sem)

    def gather_cps(ia, ib, rows_v, sem):
        return (pltpu.make_async_copy(table_hbm.at[ia],
                                      rows_v.at[pl.ds(0, GA)], sem),
                pltpu.make_async_copy(table_hbm.at[ib],
                                      rows_v.at[pl.ds(GA, GB)], sem))

    def out_cp(b, src_v, sem):
        return pltpu.make_async_copy(
            src_v.at[pl.ds(0, SEQ)], out_hbm.at[pl.ds(b * SEQ, SEQ)], sem)

    for c in range(NCHUNK):
        val = lane + jnp.full((16,), 16 * c + 1, jnp.int32)
        if c == NCHUNK - 1:
            val = jnp.where(tail_valid, val, zeros16)
        if c < 7:
            ia0[pl.ds(16 * c, 16)] = val
        else:
            ib0[pl.ds(16 * (c - 7), 16)] = val
    st_a, st_b = gather_cps(ia0, ib0, tab_v, sg0)
    st_a.start()
    st_b.start()
    st_a.wait()
    st_b.wait()

    def compute_pos(seq_v, ia, ib):
        carry = zeros16
        for c in range(NCHUNK):
            v = seq_v[pl.ds(16 * c, 16)]
            nz = v != zeros16
            if c == NCHUNK - 1:
                nz = jnp.logical_and(nz, tail_valid)
            m = jnp.where(nz, ones16, zeros16)
            s = _cumsum16(m)
            pos = (s + carry) * m
            if c < 7:
                ia[pl.ds(16 * c, 16)] = pos
            else:
                ib[pl.ds(16 * (c - 7), 16)] = pos
            carry = carry + _lanegather(s, idx_last)

    def count_nonpad(seq_v):
        acc = zeros16
        for c in range(NCHUNK):
            v = seq_v[pl.ds(16 * c, 16)]
            nz = v != zeros16
            if c == NCHUNK - 1:
                nz = jnp.logical_and(nz, tail_valid)
            acc = acc + jnp.where(nz, ones16, zeros16)
        return _allsum16(acc)[0]

    bufs = ((seq0, ia0, ib0, rows0, sin0, sg0, sout0),
            (seq1, ia1, ib1, rows1, sin1, sg1, sout1))

    in_cp(base + 0, seq0, sin0).start()
    in_cp(base + 1, seq1, sin1).start()

    def pair_loop(r2, carry_unused):
        for p in (0, 1):
            r = 2 * r2 + p
            b = base + r
            seq_v, ia, ib, rows_v, sin, sg, sout = bufs[p]
            in_cp(b, seq_v, sin).wait()
            total = count_nonpad(seq_v)

            @pl.when(r2 >= 1)
            def _():
                out_cp(b - 2, tab_v, sout).wait()

            @pl.when(total == SEQ)
            def _():
                out_cp(b, tab_v, sout).start()

            @pl.when(total != SEQ)
            def _():
                compute_pos(seq_v, ia, ib)
                ga, gb = gather_cps(ia, ib, rows_v, sg)
                ga.start()
                gb.start()
                ga.wait()
                gb.wait()
                out_cp(b, rows_v, sout).start()

            @pl.when(r2 < (ROWS_PER_W // 2) - 1)
            def _():
                in_cp(b + 2, seq_v, sin).start()

        return carry_unused

    lax.fori_loop(0, ROWS_PER_W // 2, pair_loop, jnp.int32(0))

    out_cp(base + ROWS_PER_W - 2, tab_v, sout0).wait()
    out_cp(base + ROWS_PER_W - 1, tab_v, sout1).wait()


@jax.jit
def _sc_call(seq, table):
    fn = functools.partial(
        pl.kernel,
        mesh=plsc.VectorSubcoreMesh(core_axis_name="c", subcore_axis_name="s"),
        compiler_params=pltpu.CompilerParams(use_tc_tiling_on_sc=False),
        out_type=jax.ShapeDtypeStruct((BATCH * SEQ, EMB), jnp.float32),
        scratch_types=[
            pltpu.VMEM((SEQ_PAD,), jnp.int32),
            pltpu.VMEM((SEQ_PAD,), jnp.int32),
            pltpu.VMEM((GA,), jnp.int32),
            pltpu.VMEM((GB,), jnp.int32),
            pltpu.VMEM((GA,), jnp.int32),
            pltpu.VMEM((GB,), jnp.int32),
            pltpu.VMEM((GA + GB, EMB), jnp.float32),
            pltpu.VMEM((GA + GB, EMB), jnp.float32),
            pltpu.VMEM((GA + GB, EMB), jnp.float32),
            pltpu.VMEM((16,), jnp.int32),
            pltpu.SemaphoreType.DMA,
            pltpu.SemaphoreType.DMA,
            pltpu.SemaphoreType.DMA,
            pltpu.SemaphoreType.DMA,
            pltpu.SemaphoreType.DMA,
            pltpu.SemaphoreType.DMA,
        ],
    )(_sc_body)
    return fn(seq, table)


def kernel(word_seq, position_enc_weight):
    seq = word_seq.astype(jnp.int32).reshape(-1)
    out = _sc_call(seq, position_enc_weight)
    return out.reshape(BATCH, SEQ, EMB)
